# Initial kernel scaffold; baseline (speedup 1.0000x reference)
import jax
import jax.numpy as jnp
from jax.experimental import pallas as pl


def kernel(r, offsets, idx_i, idx_j):
    raise NotImplementedError("write your pallas kernel here")


# speedup vs baseline: 5.2595x; 5.2595x over previous
import jax
import jax.numpy as jnp
from jax import lax
from jax.experimental import pallas as pl
from jax.experimental.pallas import tpu as pltpu
from jax.experimental.pallas import tpu_sc as plsc

N_NODES = 100000
N_EDGES = 6400000

NC = 2
NS = 16
L = 16
NW = NC * NS

E = 2048
G = 128
NG = E // G
NCHUNK = N_EDGES // E


def _body(rx_hbm, ry_hbm, rz_hbm, offs_hbm, idxi_hbm, idxj_hbm, out_hbm,
          rx_sh, ry_sh, rz_sh,
          idxi_v, idxj_v, offs_v,
          xi_v, yi_v, zi_v, xj_v, yj_v, zj_v, out_v):
    cid = lax.axis_index("c")
    sid = lax.axis_index("s")
    wid = sid * NC + cid

    @pl.when(sid == 0)
    def _stage():
        pltpu.sync_copy(rx_hbm, rx_sh)
        pltpu.sync_copy(ry_hbm, ry_sh)
        pltpu.sync_copy(rz_hbm, rz_sh)

    plsc.subcore_barrier()

    lanes = lax.iota(jnp.int32, L)

    def chunk_body(k):
        base = pl.multiple_of(k * E, E)
        pltpu.sync_copy(idxi_hbm.at[pl.ds(base, E)], idxi_v)
        pltpu.sync_copy(idxj_hbm.at[pl.ds(base, E)], idxj_v)
        pltpu.sync_copy(offs_hbm.at[pl.ds(base * 3, E * 3)], offs_v)

        def gather_body(j, carry):
            o = pl.multiple_of(j * G, G)
            sl = pl.ds(o, G)
            ii = idxi_v.at[sl]
            jj = idxj_v.at[sl]
            pltpu.sync_copy(rx_sh.at[ii], xi_v.at[sl])
            pltpu.sync_copy(ry_sh.at[ii], yi_v.at[sl])
            pltpu.sync_copy(rz_sh.at[ii], zi_v.at[sl])
            pltpu.sync_copy(rx_sh.at[jj], xj_v.at[sl])
            pltpu.sync_copy(ry_sh.at[jj], yj_v.at[sl])
            pltpu.sync_copy(rz_sh.at[jj], zj_v.at[sl])
            return carry

        lax.fori_loop(0, NG, gather_body, 0)

        def comp_body(g, carry):
            eb = pl.multiple_of(g * L, L)
            sl = pl.ds(eb, L)
            ev3 = (eb + lanes) * 3
            dx = xi_v[sl] - xj_v[sl] - plsc.load_gather(offs_v, [ev3])
            dy = yi_v[sl] - yj_v[sl] - plsc.load_gather(offs_v, [ev3 + 1])
            dz = zi_v[sl] - zj_v[sl] - plsc.load_gather(offs_v, [ev3 + 2])
            acc = dx * dx + dy * dy + dz * dz
            i = plsc.bitcast(acc, jnp.int32)
            y = plsc.bitcast(jnp.int32(0x5F3759DF) - (i >> 1), jnp.float32)
            y = y * (1.5 - 0.5 * acc * y * y)
            y = y * (1.5 - 0.5 * acc * y * y)
            d = jnp.where(acc > 1e-35, acc * y, 0.0)
            out_v[sl] = d
            return carry

        lax.fori_loop(0, E // L, comp_body, 0)
        pltpu.sync_copy(out_v, out_hbm.at[pl.ds(base, E)])

    nk = (NCHUNK - wid + NW - 1) // NW

    def outer(i, carry):
        chunk_body(wid + i * NW)
        return carry

    lax.fori_loop(0, nk, outer, 0)


@jax.jit
def _distances(rx, ry, rz, offsets, idx_i, idx_j):
    mesh = plsc.VectorSubcoreMesh(core_axis_name="c", subcore_axis_name="s",
                                  num_cores=NC, num_subcores=NS)
    f = pl.kernel(
        _body,
        out_type=jax.ShapeDtypeStruct((N_EDGES,), jnp.float32),
        mesh=mesh,
        compiler_params=pltpu.CompilerParams(needs_layout_passes=False),
        scratch_types=[
            pltpu.VMEM_SHARED((N_NODES,), jnp.float32),
            pltpu.VMEM_SHARED((N_NODES,), jnp.float32),
            pltpu.VMEM_SHARED((N_NODES,), jnp.float32),
            pltpu.VMEM((E,), jnp.int32),
            pltpu.VMEM((E,), jnp.int32),
            pltpu.VMEM((E * 3,), jnp.float32),
            pltpu.VMEM((E,), jnp.float32),
            pltpu.VMEM((E,), jnp.float32),
            pltpu.VMEM((E,), jnp.float32),
            pltpu.VMEM((E,), jnp.float32),
            pltpu.VMEM((E,), jnp.float32),
            pltpu.VMEM((E,), jnp.float32),
            pltpu.VMEM((E,), jnp.float32),
        ],
    )
    return f(rx, ry, rz, offsets, idx_i, idx_j)


def kernel(r, offsets, idx_i, idx_j):
    rt = r.astype(jnp.float32).T
    dij = _distances(rt[0], rt[1], rt[2],
                     offsets.astype(jnp.float32).reshape(-1),
                     idx_i.astype(jnp.int32), idx_j.astype(jnp.int32))
    return dij.reshape(N_EDGES, 1)


# speedup vs baseline: 6.1385x; 1.1671x over previous
import jax
import jax.numpy as jnp
from jax import lax
from jax.experimental import pallas as pl
from jax.experimental.pallas import tpu as pltpu
from jax.experimental.pallas import tpu_sc as plsc

N_NODES = 100000
N_EDGES = 6400000

NC = 2
NS = 16
L = 16
NW = NC * NS

E = 4000
NCHUNK = N_EDGES // E
PT = NCHUNK // NW


def _body(rx_hbm, ry_hbm, rz_hbm, offs_hbm, idxi_hbm, idxj_hbm, out_hbm,
          rx_sh, ry_sh, rz_sh,
          idxi_v, idxj_v, offs_v, xi_v, yi_v, zi_v, xj_v, yj_v, zj_v, out_v,
          sidx, soffs, sgat, sout):
    cid = lax.axis_index("c")
    sid = lax.axis_index("s")
    wid = sid * NC + cid

    @pl.when(sid == 0)
    def _stage():
        pltpu.sync_copy(rx_hbm, rx_sh)
        pltpu.sync_copy(ry_hbm, ry_sh)
        pltpu.sync_copy(rz_hbm, rz_sh)

    plsc.subcore_barrier()

    lanes = lax.iota(jnp.int32, L)

    def ebase(c):
        return pl.multiple_of((wid + c * NW) * E, E)

    def issue_idx(c, b):
        base = ebase(c)
        pltpu.async_copy(idxi_hbm.at[pl.ds(base, E)], idxi_v[b], sidx[b])
        pltpu.async_copy(idxj_hbm.at[pl.ds(base, E)], idxj_v[b], sidx[b])

    def wait_idx(b):
        pltpu.make_async_copy(idxi_hbm.at[pl.ds(0, E)], idxi_v[b],
                              sidx[b]).wait()
        pltpu.make_async_copy(idxj_hbm.at[pl.ds(0, E)], idxj_v[b],
                              sidx[b]).wait()

    def issue_offs(c, b):
        base = ebase(c)
        pltpu.async_copy(offs_hbm.at[pl.ds(base * 3, E * 3)], offs_v[b],
                         soffs[b])

    def wait_offs(b):
        pltpu.make_async_copy(offs_hbm.at[pl.ds(0, E * 3)], offs_v[b],
                              soffs[b]).wait()

    def issue_gat(b):
        pltpu.async_copy(rx_sh.at[idxi_v[b]], xi_v[b], sgat[b])
        pltpu.async_copy(ry_sh.at[idxi_v[b]], yi_v[b], sgat[b])
        pltpu.async_copy(rz_sh.at[idxi_v[b]], zi_v[b], sgat[b])
        pltpu.async_copy(rx_sh.at[idxj_v[b]], xj_v[b], sgat[b])
        pltpu.async_copy(ry_sh.at[idxj_v[b]], yj_v[b], sgat[b])
        pltpu.async_copy(rz_sh.at[idxj_v[b]], zj_v[b], sgat[b])

    def wait_gat(b):
        pltpu.make_async_copy(rx_sh.at[idxi_v[b]], xi_v[b], sgat[b]).wait()
        pltpu.make_async_copy(ry_sh.at[idxi_v[b]], yi_v[b], sgat[b]).wait()
        pltpu.make_async_copy(rz_sh.at[idxi_v[b]], zi_v[b], sgat[b]).wait()
        pltpu.make_async_copy(rx_sh.at[idxj_v[b]], xj_v[b], sgat[b]).wait()
        pltpu.make_async_copy(ry_sh.at[idxj_v[b]], yj_v[b], sgat[b]).wait()
        pltpu.make_async_copy(rz_sh.at[idxj_v[b]], zj_v[b], sgat[b]).wait()

    def issue_out(c, b):
        pltpu.async_copy(out_v[b], out_hbm.at[pl.ds(ebase(c), E)], sout[b])

    def wait_out(b):
        pltpu.make_async_copy(out_v[b], out_hbm.at[pl.ds(0, E)],
                              sout[b]).wait()

    def compute(b):
        xi, yi, zi = xi_v[b], yi_v[b], zi_v[b]
        xj, yj, zj = xj_v[b], yj_v[b], zj_v[b]
        offs, out = offs_v[b], out_v[b]

        def comp_body(g, carry):
            eb = pl.multiple_of(g * L, L)
            sl = pl.ds(eb, L)
            ev3 = (eb + lanes) * 3
            dx = xi[sl] - xj[sl] - plsc.load_gather(offs, [ev3])
            dy = yi[sl] - yj[sl] - plsc.load_gather(offs, [ev3 + 1])
            dz = zi[sl] - zj[sl] - plsc.load_gather(offs, [ev3 + 2])
            acc = dx * dx + dy * dy + dz * dz
            i = plsc.bitcast(acc, jnp.int32)
            y = plsc.bitcast(jnp.int32(0x5F3759DF) - (i >> 1), jnp.float32)
            y = y * (1.5 - 0.5 * acc * y * y)
            y = y * (1.5 - 0.5 * acc * y * y)
            out[sl] = jnp.where(acc > 1e-35, acc * y, 0.0)
            return carry

        lax.fori_loop(0, E // L, comp_body, 0, unroll=2)

    issue_idx(0, 0)
    issue_offs(0, 0)

    def step(c, b):
        wait_idx(b)
        issue_gat(b)

        @pl.when(c > 0)
        def _tail():
            wait_gat(1 - b)

            @pl.when(c + 1 < PT)
            def _():
                issue_idx(c + 1, 1 - b)

            wait_offs(1 - b)

            @pl.when(c >= 3)
            def _():
                wait_out(1 - b)

            compute(1 - b)
            issue_out(c - 1, 1 - b)

            @pl.when(c + 1 < PT)
            def _():
                issue_offs(c + 1, 1 - b)

        @pl.when(c == 0)
        def _head():
            issue_idx(1, 1)
            issue_offs(1, 1)

    def outer(i, carry):
        step(2 * i, 0)
        step(2 * i + 1, 1)
        return carry

    lax.fori_loop(0, PT // 2, outer, 0)

    wait_gat(1)
    wait_offs(1)
    wait_out(1)
    compute(1)
    issue_out(PT - 1, 1)
    wait_out(0)
    wait_out(1)


@jax.jit
def _distances(rx, ry, rz, offsets, idx_i, idx_j):
    mesh = plsc.VectorSubcoreMesh(core_axis_name="c", subcore_axis_name="s",
                                  num_cores=NC, num_subcores=NS)
    vm = lambda n, dt: pltpu.VMEM((n,), dt)
    f = pl.kernel(
        _body,
        out_type=jax.ShapeDtypeStruct((N_EDGES,), jnp.float32),
        mesh=mesh,
        compiler_params=pltpu.CompilerParams(needs_layout_passes=False),
        scratch_types=[
            pltpu.VMEM_SHARED((N_NODES,), jnp.float32),
            pltpu.VMEM_SHARED((N_NODES,), jnp.float32),
            pltpu.VMEM_SHARED((N_NODES,), jnp.float32),
            [vm(E, jnp.int32)] * 2,
            [vm(E, jnp.int32)] * 2,
            [vm(E * 3, jnp.float32)] * 2,
            [vm(E, jnp.float32)] * 2,
            [vm(E, jnp.float32)] * 2,
            [vm(E, jnp.float32)] * 2,
            [vm(E, jnp.float32)] * 2,
            [vm(E, jnp.float32)] * 2,
            [vm(E, jnp.float32)] * 2,
            [vm(E, jnp.float32)] * 2,
            [pltpu.SemaphoreType.DMA] * 2,
            [pltpu.SemaphoreType.DMA] * 2,
            [pltpu.SemaphoreType.DMA] * 2,
            [pltpu.SemaphoreType.DMA] * 2,
        ],
    )
    return f(rx, ry, rz, offsets, idx_i, idx_j)


def kernel(r, offsets, idx_i, idx_j):
    rt = r.astype(jnp.float32).T
    dij = _distances(rt[0], rt[1], rt[2],
                     offsets.astype(jnp.float32).reshape(-1),
                     idx_i.astype(jnp.int32), idx_j.astype(jnp.int32))
    return dij.reshape(N_EDGES, 1)


# speedup vs baseline: 105.9158x; 17.2543x over previous
import jax
import jax.numpy as jnp
from jax import lax
from jax.experimental import pallas as pl
from jax.experimental.pallas import tpu as pltpu
from jax.experimental.pallas import tpu_sc as plsc

N_NODES = 100000
N_EDGES = 6400000

NC = 2
NS = 16
L = 16
NW = NC * NS

E = 4000
NCHUNK = N_EDGES // E
PT = NCHUNK // NW


def _body(rx_hbm, ry_hbm, rz_hbm, ox_hbm, oy_hbm, oz_hbm,
          idxi_hbm, idxj_hbm, out_hbm,
          rx_sh, ry_sh, rz_sh,
          idxi_v, idxj_v, ox_v, oy_v, oz_v,
          xi_v, yi_v, zi_v, xj_v, yj_v, zj_v, out_v,
          sidx, soffs, sgat, sout):
    cid = lax.axis_index("c")
    sid = lax.axis_index("s")
    wid = sid * NC + cid

    @pl.when(sid == 0)
    def _stage():
        pltpu.sync_copy(rx_hbm, rx_sh)
        pltpu.sync_copy(ry_hbm, ry_sh)
        pltpu.sync_copy(rz_hbm, rz_sh)

    plsc.subcore_barrier()

    def ebase(c):
        return pl.multiple_of((wid + c * NW) * E, E)

    def issue_idx(c, b):
        base = ebase(c)
        pltpu.async_copy(idxi_hbm.at[pl.ds(base, E)], idxi_v[b], sidx[b])
        pltpu.async_copy(idxj_hbm.at[pl.ds(base, E)], idxj_v[b], sidx[b])

    def wait_idx(b):
        pltpu.make_async_copy(idxi_hbm.at[pl.ds(0, E)], idxi_v[b],
                              sidx[b]).wait()
        pltpu.make_async_copy(idxj_hbm.at[pl.ds(0, E)], idxj_v[b],
                              sidx[b]).wait()

    def issue_offs(c, b):
        base = ebase(c)
        pltpu.async_copy(ox_hbm.at[pl.ds(base, E)], ox_v[b], soffs[b])
        pltpu.async_copy(oy_hbm.at[pl.ds(base, E)], oy_v[b], soffs[b])
        pltpu.async_copy(oz_hbm.at[pl.ds(base, E)], oz_v[b], soffs[b])

    def wait_offs(b):
        pltpu.make_async_copy(ox_hbm.at[pl.ds(0, E)], ox_v[b],
                              soffs[b]).wait()
        pltpu.make_async_copy(oy_hbm.at[pl.ds(0, E)], oy_v[b],
                              soffs[b]).wait()
        pltpu.make_async_copy(oz_hbm.at[pl.ds(0, E)], oz_v[b],
                              soffs[b]).wait()

    def issue_gat(b):
        pltpu.async_copy(rx_sh.at[idxi_v[b]], xi_v[b], sgat[b])
        pltpu.async_copy(ry_sh.at[idxi_v[b]], yi_v[b], sgat[b])
        pltpu.async_copy(rz_sh.at[idxi_v[b]], zi_v[b], sgat[b])
        pltpu.async_copy(rx_sh.at[idxj_v[b]], xj_v[b], sgat[b])
        pltpu.async_copy(ry_sh.at[idxj_v[b]], yj_v[b], sgat[b])
        pltpu.async_copy(rz_sh.at[idxj_v[b]], zj_v[b], sgat[b])

    def wait_gat(b):
        pltpu.make_async_copy(rx_sh.at[idxi_v[b]], xi_v[b], sgat[b]).wait()
        pltpu.make_async_copy(ry_sh.at[idxi_v[b]], yi_v[b], sgat[b]).wait()
        pltpu.make_async_copy(rz_sh.at[idxi_v[b]], zi_v[b], sgat[b]).wait()
        pltpu.make_async_copy(rx_sh.at[idxj_v[b]], xj_v[b], sgat[b]).wait()
        pltpu.make_async_copy(ry_sh.at[idxj_v[b]], yj_v[b], sgat[b]).wait()
        pltpu.make_async_copy(rz_sh.at[idxj_v[b]], zj_v[b], sgat[b]).wait()

    def issue_out(c, b):
        pltpu.async_copy(out_v[b], out_hbm.at[pl.ds(ebase(c), E)], sout[b])

    def wait_out(b):
        pltpu.make_async_copy(out_v[b], out_hbm.at[pl.ds(0, E)],
                              sout[b]).wait()

    def compute(b):
        xi, yi, zi = xi_v[b], yi_v[b], zi_v[b]
        xj, yj, zj = xj_v[b], yj_v[b], zj_v[b]
        ox, oy, oz = ox_v[b], oy_v[b], oz_v[b]
        out = out_v[b]

        def comp_body(g, carry):
            eb = pl.multiple_of(g * L, L)
            sl = pl.ds(eb, L)
            dx = xi[sl] - xj[sl] - ox[sl]
            dy = yi[sl] - yj[sl] - oy[sl]
            dz = zi[sl] - zj[sl] - oz[sl]
            acc = dx * dx + dy * dy + dz * dz
            i = plsc.bitcast(acc, jnp.int32)
            y = plsc.bitcast(jnp.int32(0x5F3759DF) - (i >> 1), jnp.float32)
            y = y * (1.5 - 0.5 * acc * y * y)
            y = y * (1.5 - 0.5 * acc * y * y)
            out[sl] = jnp.where(acc > 1e-35, acc * y, 0.0)
            return carry

        lax.fori_loop(0, E // L, comp_body, 0, unroll=2)

    issue_idx(0, 0)
    issue_offs(0, 0)

    def step(c, b):
        wait_idx(b)
        issue_gat(b)

        @pl.when(c > 0)
        def _tail():
            wait_gat(1 - b)

            @pl.when(c + 1 < PT)
            def _():
                issue_idx(c + 1, 1 - b)

            wait_offs(1 - b)

            @pl.when(c >= 3)
            def _():
                wait_out(1 - b)

            compute(1 - b)
            issue_out(c - 1, 1 - b)

            @pl.when(c + 1 < PT)
            def _():
                issue_offs(c + 1, 1 - b)

        @pl.when(c == 0)
        def _head():
            issue_idx(1, 1)
            issue_offs(1, 1)

    def outer(i, carry):
        step(2 * i, 0)
        step(2 * i + 1, 1)
        return carry

    lax.fori_loop(0, PT // 2, outer, 0)

    wait_gat(1)
    wait_offs(1)
    wait_out(1)
    compute(1)
    issue_out(PT - 1, 1)
    wait_out(0)
    wait_out(1)


@jax.jit
def _distances(rx, ry, rz, ox, oy, oz, idx_i, idx_j):
    mesh = plsc.VectorSubcoreMesh(core_axis_name="c", subcore_axis_name="s",
                                  num_cores=NC, num_subcores=NS)
    vm = lambda n, dt: pltpu.VMEM((n,), dt)
    f = pl.kernel(
        _body,
        out_type=jax.ShapeDtypeStruct((N_EDGES,), jnp.float32),
        mesh=mesh,
        compiler_params=pltpu.CompilerParams(needs_layout_passes=False),
        scratch_types=[
            pltpu.VMEM_SHARED((N_NODES,), jnp.float32),
            pltpu.VMEM_SHARED((N_NODES,), jnp.float32),
            pltpu.VMEM_SHARED((N_NODES,), jnp.float32),
            [vm(E, jnp.int32)] * 2,
            [vm(E, jnp.int32)] * 2,
            [vm(E, jnp.float32)] * 2,
            [vm(E, jnp.float32)] * 2,
            [vm(E, jnp.float32)] * 2,
            [vm(E, jnp.float32)] * 2,
            [vm(E, jnp.float32)] * 2,
            [vm(E, jnp.float32)] * 2,
            [vm(E, jnp.float32)] * 2,
            [vm(E, jnp.float32)] * 2,
            [vm(E, jnp.float32)] * 2,
            [vm(E, jnp.float32)] * 2,
            [pltpu.SemaphoreType.DMA] * 2,
            [pltpu.SemaphoreType.DMA] * 2,
            [pltpu.SemaphoreType.DMA] * 2,
            [pltpu.SemaphoreType.DMA] * 2,
        ],
    )
    return f(rx, ry, rz, ox, oy, oz, idx_i, idx_j)


def kernel(r, offsets, idx_i, idx_j):
    rt = r.astype(jnp.float32).T
    ot = offsets.astype(jnp.float32).T
    dij = _distances(rt[0], rt[1], rt[2], ot[0], ot[1], ot[2],
                     idx_i.astype(jnp.int32), idx_j.astype(jnp.int32))
    return dij.reshape(N_EDGES, 1)
